# Initial kernel scaffold; baseline (speedup 1.0000x reference)
import functools

import jax
import jax.numpy as jnp
from jax import lax
from jax.experimental import pallas as pl
from jax.experimental.pallas import tpu as pltpu
from jax.experimental.pallas import tpu_sc as plsc

A = 50000
N = 32
B = 4
ROW = 16
CHUNK = 50
NCHUNKS = A // CHUNK
NW = 32
OPB = CHUNK * N * 3

_U = [[(16 * s + k) // 3 for k in range(16)] for s in range(6)]
_V = [[(16 * s + k) % 3 for k in range(16)] for s in range(6)]


def _sc_call(table, neigh_flat):
    mesh = plsc.VectorSubcoreMesh(core_axis_name="c", subcore_axis_name="s")

    @functools.partial(
        pl.kernel,
        out_type=jax.ShapeDtypeStruct((B * A * N * 3,), jnp.float32),
        mesh=mesh,
        scratch_types=[
            pltpu.VMEM((CHUNK * N,), jnp.int32),
            pltpu.VMEM((CHUNK * N, ROW), jnp.float32),
            pltpu.VMEM((CHUNK, ROW), jnp.float32),
            pltpu.VMEM((B, OPB), jnp.float32),
            pltpu.SemaphoreType.DMA,
        ],
    )
    def k(table_hbm, neigh_hbm, out_hbm, idx_v, gath_v, cent_v, ob, sem):
        cid = lax.axis_index("c")
        sid = lax.axis_index("s")
        wid = sid * 2 + cid
        nmine = jnp.where(wid < NCHUNKS - (NCHUNKS // NW) * NW, 1, 0) + NCHUNKS // NW

        uv = [jnp.array(_U[s], dtype=jnp.int32) for s in range(6)]
        cv = [[jnp.array([3 * b + v for v in _V[s]], dtype=jnp.int32)
               for s in range(6)] for b in range(B)]

        def chunk_body(t, carry):
            chunk = wid + NW * t
            pltpu.sync_copy(neigh_hbm.at[pl.ds(chunk * CHUNK * N, CHUNK * N)],
                            idx_v)
            pltpu.async_copy(table_hbm.at[idx_v], gath_v, sem).wait()
            pltpu.sync_copy(table_hbm.at[pl.ds(chunk * CHUNK, CHUNK)], cent_v)

            def atom_body(i, c2):
                irow = jnp.broadcast_to(i, (16,)).astype(jnp.int32)
                for s in range(6):
                    row_vec = i * N + uv[s]
                    for b in range(B):
                        g = plsc.load_gather(gath_v, [row_vec, cv[b][s]])
                        cb = plsc.load_gather(cent_v, [irow, cv[b][s]])
                        ob[b, pl.ds(96 * i + 16 * s, 16)] = g - cb
                return c2

            lax.fori_loop(0, CHUNK, atom_body, 0)
            for b in range(B):
                pltpu.sync_copy(
                    ob.at[b],
                    out_hbm.at[pl.ds(b * A * N * 3 + chunk * OPB, OPB)])
            return carry

        lax.fori_loop(0, nmine, chunk_body, 0)

    return k(table, neigh_flat)


def kernel(atoms, neighbors):
    table = jnp.transpose(atoms, (1, 0, 2)).reshape(A, 3 * B)
    table = jnp.pad(table, ((0, 0), (0, ROW - 3 * B)))
    neigh_flat = neighbors.astype(jnp.int32).reshape(-1)
    out = _sc_call(table, neigh_flat)
    return out.reshape(B, A, N, 3)


# speedup vs baseline: 10.3950x; 10.3950x over previous
import functools

import jax
import jax.numpy as jnp
from jax import lax
from jax.experimental import pallas as pl
from jax.experimental.pallas import tpu as pltpu
from jax.experimental.pallas import tpu_sc as plsc

A = 50000
N = 32
B = 4
ROW = 16
CHUNK = 80
NCHUNKS = A // CHUNK
NW = 32
NQ = CHUNK // 16


def _sc_call(table, neigh_flat):
    mesh = plsc.VectorSubcoreMesh(core_axis_name="c", subcore_axis_name="s")

    @functools.partial(
        pl.kernel,
        out_type=jax.ShapeDtypeStruct((B, 3, N, A), jnp.float32),
        mesh=mesh,
        compiler_params=pltpu.CompilerParams(
            use_tc_tiling_on_sc=False, needs_layout_passes=False),
        scratch_types=[
            pltpu.VMEM((CHUNK * N,), jnp.int32),
            pltpu.VMEM((CHUNK * N, ROW), jnp.float32),
            pltpu.VMEM((CHUNK, ROW), jnp.float32),
            pltpu.VMEM((B * 3, N, CHUNK), jnp.float32),
            pltpu.SemaphoreType.DMA,
        ],
    )
    def k(table_hbm, neigh_hbm, out_hbm, idx_v, gath_v, cent_v, ob, sem):
        cid = lax.axis_index("c")
        sid = lax.axis_index("s")
        wid = sid * 2 + cid
        nmine = jnp.where(wid < NCHUNKS - (NCHUNKS // NW) * NW, 1, 0) + NCHUNKS // NW

        lane = lax.iota(jnp.int32, 16)
        lane_g = lane * N
        colv = [[lane * 0 + (3 * b + c) for c in range(3)] for b in range(B)]

        def chunk_body(t, carry):
            chunk = wid + NW * t
            a0 = chunk * CHUNK
            pltpu.sync_copy(neigh_hbm.at[pl.ds(a0 * N, CHUNK * N)], idx_v)
            pltpu.async_copy(table_hbm.at[idx_v], gath_v, sem).wait()
            pltpu.sync_copy(table_hbm.at[pl.ds(a0, CHUNK)], cent_v)

            def q_body(q, c2):
                crow = lane + q * 16
                cvec = [[plsc.load_gather(cent_v, [crow, colv[b][c]])
                         for c in range(3)] for b in range(B)]

                def n_body(n, c3):
                    grow = lane_g + (q * 16 * N + n)
                    for b in range(B):
                        for c in range(3):
                            g = plsc.load_gather(gath_v, [grow, colv[b][c]])
                            ob[b * 3 + c, n, pl.ds(q * 16, 16)] = \
                                g - cvec[b][c]
                    return c3

                lax.fori_loop(0, N, n_body, 0)
                return c2

            lax.fori_loop(0, NQ, q_body, 0)
            for b in range(B):
                for c in range(3):
                    pltpu.sync_copy(
                        ob.at[b * 3 + c],
                        out_hbm.at[b, c, :, pl.ds(a0, CHUNK)])
            return carry

        lax.fori_loop(0, nmine, chunk_body, 0)

    return k(table, neigh_flat)


def kernel(atoms, neighbors):
    table = jnp.transpose(atoms, (1, 0, 2)).reshape(A, 3 * B)
    table = jnp.pad(table, ((0, 0), (0, ROW - 3 * B)))
    neigh_flat = neighbors.astype(jnp.int32).reshape(-1)
    out = _sc_call(table, neigh_flat)
    return out.transpose(0, 3, 2, 1)


# speedup vs baseline: 10.5393x; 1.0139x over previous
import functools

import jax
import jax.numpy as jnp
from jax import lax
from jax.experimental import pallas as pl
from jax.experimental.pallas import tpu as pltpu
from jax.experimental.pallas import tpu_sc as plsc

A = 50000
N = 32
B = 4
ROW = 16
CHUNK = 80
NCHUNKS = A // CHUNK
NW = 32
NQ = CHUNK // 16


def _sc_call(table, neigh_flat):
    mesh = plsc.VectorSubcoreMesh(core_axis_name="c", subcore_axis_name="s")

    @functools.partial(
        pl.kernel,
        out_type=jax.ShapeDtypeStruct((B * 3, N, A), jnp.float32),
        mesh=mesh,
        compiler_params=pltpu.CompilerParams(
            use_tc_tiling_on_sc=False, needs_layout_passes=False),
        scratch_types=[
            pltpu.VMEM((CHUNK * N,), jnp.int32),
            pltpu.VMEM((CHUNK * N, ROW), jnp.float32),
            pltpu.VMEM((CHUNK, ROW), jnp.float32),
            pltpu.VMEM((B * 3, N, CHUNK), jnp.float32),
            pltpu.SemaphoreType.DMA,
        ],
    )
    def k(table_hbm, neigh_hbm, out_hbm, idx_v, gath_v, cent_v, ob, sem):
        cid = lax.axis_index("c")
        sid = lax.axis_index("s")
        wid = sid * 2 + cid
        nmine = jnp.where(wid < NCHUNKS - (NCHUNKS // NW) * NW, 1, 0) + NCHUNKS // NW

        lane = lax.iota(jnp.int32, 16)
        lane_g = lane * N
        colv = [[lane * 0 + (3 * b + c) for c in range(3)] for b in range(B)]

        def chunk_body(t, carry):
            chunk = wid + NW * t
            a0 = chunk * CHUNK
            pltpu.sync_copy(neigh_hbm.at[pl.ds(a0 * N, CHUNK * N)], idx_v)
            pltpu.async_copy(table_hbm.at[idx_v], gath_v, sem).wait()
            pltpu.sync_copy(table_hbm.at[pl.ds(a0, CHUNK)], cent_v)

            def q_body(q, c2):
                crow = lane + q * 16
                cvec = [[plsc.load_gather(cent_v, [crow, colv[b][c]])
                         for c in range(3)] for b in range(B)]

                def n_body(n, c3):
                    grow = lane_g + (q * 16 * N + n)
                    for b in range(B):
                        for c in range(3):
                            g = plsc.load_gather(gath_v, [grow, colv[b][c]])
                            ob[b * 3 + c, n, pl.ds(q * 16, 16)] = \
                                g - cvec[b][c]
                    return c3

                lax.fori_loop(0, N, n_body, 0)
                return c2

            lax.fori_loop(0, NQ, q_body, 0)
            pltpu.sync_copy(ob, out_hbm.at[:, :, pl.ds(a0, CHUNK)])
            return carry

        lax.fori_loop(0, nmine, chunk_body, 0)

    return k(table, neigh_flat)


def kernel(atoms, neighbors):
    table = jnp.transpose(atoms, (1, 0, 2)).reshape(A, 3 * B)
    table = jnp.pad(table, ((0, 0), (0, ROW - 3 * B)))
    neigh_flat = neighbors.astype(jnp.int32).reshape(-1)
    out = _sc_call(table, neigh_flat)
    return out.reshape(B, 3, N, A).transpose(0, 3, 2, 1)


# speedup vs baseline: 11.5303x; 1.0940x over previous
import functools

import jax
import jax.numpy as jnp
from jax import lax
from jax.experimental import pallas as pl
from jax.experimental.pallas import tpu as pltpu
from jax.experimental.pallas import tpu_sc as plsc

A = 50000
N = 32
B = 4
ROW = 16
CHUNK = 80
NCHUNKS = A // CHUNK
NW = 32
NQ = CHUNK // 16
NFULL = NCHUNKS // NW
NREM = NCHUNKS - NFULL * NW


def _sc_call(table, neigh_flat):
    mesh = plsc.VectorSubcoreMesh(core_axis_name="c", subcore_axis_name="s")

    @functools.partial(
        pl.kernel,
        out_type=jax.ShapeDtypeStruct((B * 3, N, A), jnp.float32),
        mesh=mesh,
        compiler_params=pltpu.CompilerParams(
            use_tc_tiling_on_sc=False, needs_layout_passes=False),
        scratch_types=[
            pltpu.VMEM((2, CHUNK * N), jnp.int32),
            pltpu.VMEM((2, CHUNK * N, ROW), jnp.float32),
            pltpu.VMEM((2, CHUNK, ROW), jnp.float32),
            pltpu.VMEM((B * 3, N, CHUNK), jnp.float32),
            pltpu.SemaphoreType.DMA,
            pltpu.SemaphoreType.DMA,
            pltpu.SemaphoreType.DMA,
            pltpu.SemaphoreType.DMA,
        ],
    )
    def k(table_hbm, neigh_hbm, out_hbm, idx_v, gath_v, cent_v, ob,
          gs0, gs1, cs0, cs1):
        cid = lax.axis_index("c")
        sid = lax.axis_index("s")
        wid = sid * 2 + cid

        lane = lax.iota(jnp.int32, 16)
        lane_g = lane * N
        colv = [[lane * 0 + (3 * b + c) for c in range(3)] for b in range(B)]
        gsem = (gs0, gs1)
        csem = (cs0, cs1)

        def fetch(t, s):
            chunk = wid + NW * t
            a0 = chunk * CHUNK
            pltpu.sync_copy(neigh_hbm.at[pl.ds(a0 * N, CHUNK * N)],
                            idx_v.at[s])
            hg = pltpu.async_copy(table_hbm.at[idx_v.at[s]], gath_v.at[s],
                                  gsem[s])
            hc = pltpu.async_copy(table_hbm.at[pl.ds(a0, CHUNK)],
                                  cent_v.at[s], csem[s])
            return hg, hc

        def compute(t, s):
            chunk = wid + NW * t
            a0 = chunk * CHUNK
            gath = gath_v.at[s]
            cent = cent_v.at[s]

            def q_body(q, c2):
                crow = lane + q * 16
                cvec = [[plsc.load_gather(cent, [crow, colv[b][c]])
                         for c in range(3)] for b in range(B)]

                def n_body(n, c3):
                    grow = lane_g + (q * 16 * N + n)
                    for b in range(B):
                        for c in range(3):
                            g = plsc.load_gather(gath, [grow, colv[b][c]])
                            ob[b * 3 + c, n, pl.ds(q * 16, 16)] = \
                                g - cvec[b][c]
                    return c3

                lax.fori_loop(0, N, n_body, 0)
                return c2

            lax.fori_loop(0, NQ, q_body, 0)
            pltpu.sync_copy(ob, out_hbm.at[:, :, pl.ds(a0, CHUNK)])

        hand = fetch(0, 0)
        for t in range(NFULL):
            s = t % 2
            nxt = None
            if t + 1 < NFULL:
                nxt = fetch(t + 1, 1 - s)
            hand[0].wait()
            hand[1].wait()
            compute(t, s)
            hand = nxt

        @pl.when(wid < NREM)
        def _():
            hg, hc = fetch(NFULL, NFULL % 2)
            hg.wait()
            hc.wait()
            compute(NFULL, NFULL % 2)

    return k(table, neigh_flat)


def kernel(atoms, neighbors):
    table = jnp.transpose(atoms, (1, 0, 2)).reshape(A, 3 * B)
    table = jnp.pad(table, ((0, 0), (0, ROW - 3 * B)))
    neigh_flat = neighbors.astype(jnp.int32).reshape(-1)
    out = _sc_call(table, neigh_flat)
    return out.reshape(B, 3, N, A).transpose(0, 3, 2, 1)
